# baseline (device time: 21395 ns/iter reference)
import jax
import jax.numpy as jnp
from jax import lax
from jax.experimental import pallas as pl
from jax.experimental.pallas import tpu as pltpu

N_DEV = 4
EPS = 1e-5


def kernel(x, gamma, beta):
    m, n_loc = x.shape
    n_glob = n_loc * N_DEV
    gamma2 = gamma.reshape(1, n_loc)
    beta2 = beta.reshape(1, n_loc)

    def body(x_ref, g_ref, b_ref, out_ref, comm_ref, send_sems, recv_sems):
        my_pos = lax.axis_index("i")

        barrier = pltpu.get_barrier_semaphore()
        for d in range(1, N_DEV):
            peer = lax.rem(my_pos + d, N_DEV)
            pl.semaphore_signal(
                barrier, inc=1,
                device_id=(peer,), device_id_type=pl.DeviceIdType.MESH,
            )
        pl.semaphore_wait(barrier, N_DEV - 1)

        xs = x_ref[:, :].astype(jnp.float32)
        comm_ref[0, :, 0:1] = jnp.sum(xs, axis=1, keepdims=True)
        comm_ref[0, :, 1:2] = jnp.sum(xs * xs, axis=1, keepdims=True)

        rdmas = []
        for d in range(1, N_DEV):
            peer = lax.rem(my_pos + d, N_DEV)
            rdma = pltpu.make_async_remote_copy(
                src_ref=comm_ref.at[0],
                dst_ref=comm_ref.at[d],
                send_sem=send_sems.at[d],
                recv_sem=recv_sems.at[d],
                device_id=(peer,),
                device_id_type=pl.DeviceIdType.MESH,
            )
            rdma.start()
            rdmas.append(rdma)
        for r in rdmas:
            r.wait_send()
        for r in rdmas:
            r.wait_recv()

        total = (comm_ref[0, :, :] + comm_ref[1, :, :]) + (
            comm_ref[2, :, :] + comm_ref[3, :, :]
        )
        mean = total[:, 0:1] / n_glob
        var = total[:, 1:2] / n_glob - mean * mean
        inv = lax.rsqrt(var + EPS)
        g = g_ref[0:1, :].astype(jnp.float32)
        b = b_ref[0:1, :].astype(jnp.float32)
        out_ref[:, :] = (g * ((xs - mean) * inv) + b).astype(out_ref.dtype)

    return pl.pallas_call(
        body,
        out_shape=jax.ShapeDtypeStruct((m, n_loc), x.dtype),
        in_specs=[pl.BlockSpec(memory_space=pltpu.VMEM)] * 3,
        out_specs=pl.BlockSpec(memory_space=pltpu.VMEM),
        scratch_shapes=[
            pltpu.VMEM((N_DEV, m, 2), jnp.float32),
            pltpu.SemaphoreType.DMA((N_DEV,)),
            pltpu.SemaphoreType.DMA((N_DEV,)),
        ],
        compiler_params=pltpu.CompilerParams(collective_id=0),
    )(x, gamma2, beta2)


# device time: 9967 ns/iter; 2.1466x vs baseline; 2.1466x over previous
import jax
import jax.numpy as jnp
from jax import lax
from jax.experimental import pallas as pl
from jax.experimental.pallas import tpu as pltpu

N_DEV = 4
EPS = 1e-5


def kernel(x, gamma, beta):
    m, n_loc = x.shape
    n_glob = n_loc * N_DEV
    sub = m // 128
    gamma2 = gamma.reshape(1, n_loc)
    beta2 = beta.reshape(1, n_loc)

    def body(x_ref, g_ref, b_ref, out_ref, comm_ref, send_sems, recv_sems):
        my_pos = lax.axis_index("i")

        barrier = pltpu.get_barrier_semaphore()
        for d in range(1, N_DEV):
            peer = lax.rem(my_pos + d, N_DEV)
            pl.semaphore_signal(
                barrier, inc=1,
                device_id=(peer,), device_id_type=pl.DeviceIdType.MESH,
            )

        xs = x_ref[:, :].astype(jnp.float32)
        x3 = xs.reshape(sub, 128, n_loc)
        comm_ref[0, 0:sub, :] = jnp.sum(x3, axis=-1)
        comm_ref[0, sub : 2 * sub, :] = jnp.sum(x3 * x3, axis=-1)

        pl.semaphore_wait(barrier, N_DEV - 1)

        rdmas = []
        for d in range(1, N_DEV):
            peer = lax.rem(my_pos + d, N_DEV)
            rdma = pltpu.make_async_remote_copy(
                src_ref=comm_ref.at[0],
                dst_ref=comm_ref.at[d],
                send_sem=send_sems.at[d],
                recv_sem=recv_sems.at[d],
                device_id=(peer,),
                device_id_type=pl.DeviceIdType.MESH,
            )
            rdma.start()
            rdmas.append(rdma)
        for r in rdmas:
            r.wait_send()
        for r in rdmas:
            r.wait_recv()

        total = (comm_ref[0, :, :] + comm_ref[1, :, :]) + (
            comm_ref[2, :, :] + comm_ref[3, :, :]
        )
        mean8 = total[0:sub, :] * (1.0 / n_glob)
        var8 = total[sub : 2 * sub, :] * (1.0 / n_glob) - mean8 * mean8
        inv8 = lax.rsqrt(var8 + EPS)
        g3 = g_ref[0:1, :].astype(jnp.float32).reshape(1, 1, n_loc)
        b3 = b_ref[0:1, :].astype(jnp.float32).reshape(1, 1, n_loc)
        out3 = g3 * ((x3 - mean8[:, :, None]) * inv8[:, :, None]) + b3
        out_ref[:, :] = out3.reshape(m, n_loc).astype(out_ref.dtype)

    return pl.pallas_call(
        body,
        out_shape=jax.ShapeDtypeStruct((m, n_loc), x.dtype),
        in_specs=[pl.BlockSpec(memory_space=pltpu.VMEM)] * 3,
        out_specs=pl.BlockSpec(memory_space=pltpu.VMEM),
        scratch_shapes=[
            pltpu.VMEM((N_DEV, 2 * sub, 128), jnp.float32),
            pltpu.SemaphoreType.DMA((N_DEV,)),
            pltpu.SemaphoreType.DMA((N_DEV,)),
        ],
        compiler_params=pltpu.CompilerParams(collective_id=0),
    )(x, gamma2, beta2)
